# baseline (speedup 1.0000x reference)
import functools

import jax
import jax.numpy as jnp
from jax import lax
from jax.experimental import pallas as pl
from jax.experimental.pallas import tpu as pltpu
from jax.experimental.pallas import tpu_sc as plsc

N = 10000
NP = 10240
H = 64
RB = 512
GRID = NP // RB
CHUNK = 128
NC = 2
NS = 16
NW = NC * NS
RPS = NP // NS


HR = NP // 16


def _counts_body(dst_hbm, src_hbm, ident_hbm, degp, cntp,
                 dsti_v, srci_v, ident_v, dh, ch, dacc, cacc):
    cid = lax.axis_index("c")
    sid = lax.axis_index("s")
    w = cid * NS + sid
    cpw = dst_hbm.shape[0] // NW
    zv = jnp.zeros((16,), jnp.float32)

    def zero(r, c):
        dh[r] = zv
        ch[r] = zv
        return c

    lax.fori_loop(0, HR, zero, 0)

    @pl.when(sid == 0)
    def _():
        pltpu.sync_copy(dh, dacc)
        pltpu.sync_copy(ch, cacc)

    pltpu.sync_copy(ident_hbm, ident_v)
    pltpu.sync_copy(dst_hbm.at[pl.ds(w * cpw, cpw)], dsti_v)
    pltpu.sync_copy(src_hbm.at[pl.ds(w * cpw, cpw)], srci_v)
    plsc.subcore_barrier()

    ones = jnp.ones((16,), jnp.float32)

    def hist(k, c):
        for u in range(CHUNK // 16):
            di = dsti_v[k, pl.ds(u * 16, 16)]
            plsc.addupdate_scatter(dh, [di >> 4, di & 15], ones)
            si = srci_v[k, pl.ds(u * 16, 16)]
            plsc.addupdate_scatter(ch, [si >> 4, si & 15], ones)
        return c

    lax.fori_loop(0, cpw, hist, 0)
    for k in range(HR // CHUNK):
        s = pl.ds(k * CHUNK, CHUNK)
        pltpu.sync_copy(dh.at[s], dacc.at[ident_v.at[k]], add=True)
        pltpu.sync_copy(ch.at[s], cacc.at[ident_v.at[k]], add=True)
    plsc.subcore_barrier()
    rps = HR // NS
    s = pl.ds(sid * rps, rps)
    pltpu.sync_copy(dacc.at[s], degp.at[cid, s])
    pltpu.sync_copy(cacc.at[s], cntp.at[cid, s])


@functools.lru_cache(maxsize=None)
def _make_counts(ec_rows):
    mesh = plsc.VectorSubcoreMesh(core_axis_name="c", subcore_axis_name="s",
                                  num_cores=NC, num_subcores=NS)
    cpw = ec_rows // NW
    return pl.kernel(
        _counts_body,
        out_type=(jax.ShapeDtypeStruct((NC, HR, 16), jnp.float32),
                  jax.ShapeDtypeStruct((NC, HR, 16), jnp.float32)),
        mesh=mesh,
        scratch_types=[
            pltpu.VMEM((cpw, CHUNK), jnp.int32),
            pltpu.VMEM((cpw, CHUNK), jnp.int32),
            pltpu.VMEM((HR // CHUNK, CHUNK), jnp.int32),
            pltpu.VMEM((HR, 16), jnp.float32),
            pltpu.VMEM((HR, 16), jnp.float32),
            pltpu.VMEM_SHARED((HR, 16), jnp.float32),
            pltpu.VMEM_SHARED((HR, 16), jnp.float32),
        ],
        compiler_params=pltpu.CompilerParams(use_tc_tiling_on_sc=False,
                                             needs_layout_passes=False),
    )


BUFN = 8
LEAD = BUFN // 2


def _agg_body(table_hbm, gi_hbm, si_hbm, zeros_hbm, aggp, gi_v, si_v, *sc):
    bufs = sc[:BUFN]
    acc = sc[BUFN]
    gsem = sc[BUFN + 1:2 * BUFN + 1]
    ssem = sc[2 * BUFN + 1:]
    cid = lax.axis_index("c")
    sid = lax.axis_index("s")
    w = cid * NS + sid
    cpw = gi_hbm.shape[0] // NW
    r0 = sid * RPS
    pltpu.sync_copy(zeros_hbm.at[pl.ds(r0, RPS)], acc.at[pl.ds(r0, RPS)])
    pltpu.sync_copy(gi_hbm.at[pl.ds(w * cpw, cpw)], gi_v)
    pltpu.sync_copy(si_hbm.at[pl.ds(w * cpw, cpw)], si_v)
    plsc.subcore_barrier()

    def gather(j, b):
        pltpu.async_copy(table_hbm.at[gi_v.at[j]], bufs[b], gsem[b])

    def gather_wait(j, b):
        pltpu.make_async_copy(table_hbm.at[gi_v.at[j]], bufs[b], gsem[b]).wait()

    def scat(j, b):
        pltpu.async_copy(bufs[b], acc.at[si_v.at[j]], ssem[b], add=True)

    def scat_wait(j, b):
        pltpu.make_async_copy(bufs[b], acc.at[si_v.at[j]], ssem[b]).wait()

    for b in range(LEAD):
        gather(b, b)
    for b in range(BUFN):
        gather_wait(b, b)
        scat(b, b)
        if b >= LEAD:
            scat_wait(b - LEAD, (b + LEAD) % BUFN)
        gather(b + LEAD, (b + LEAD) % BUFN)

    def body(jr, c):
        j0 = jr * BUFN
        for b in range(BUFN):
            jb = j0 + b
            gather_wait(jb, b)
            scat(jb, b)
            scat_wait(jb - LEAD, (b + LEAD) % BUFN)
            gather(lax.rem(jb + LEAD, cpw), (b + LEAD) % BUFN)
        return c

    lax.fori_loop(1, cpw // BUFN, body, 0)
    for t in range(LEAD):
        scat_wait(cpw - LEAD + t, (cpw - LEAD + t) % BUFN)
        gather_wait(t, t)
    plsc.subcore_barrier()
    pltpu.sync_copy(acc.at[pl.ds(r0, RPS)], aggp.at[cid, pl.ds(r0, RPS)])


@functools.lru_cache(maxsize=None)
def _make_agg(ec_rows, w=H):
    mesh = plsc.VectorSubcoreMesh(core_axis_name="c", subcore_axis_name="s",
                                  num_cores=NC, num_subcores=NS)
    cpw = ec_rows // NW
    return pl.kernel(
        _agg_body,
        out_type=jax.ShapeDtypeStruct((NC, NP, w), jnp.float32),
        mesh=mesh,
        scratch_types=(
            [pltpu.VMEM((cpw, CHUNK), jnp.int32)] * 2
            + [pltpu.VMEM((CHUNK, w), jnp.float32)] * BUFN
            + [pltpu.VMEM_SHARED((NP, w), jnp.float32)]
            + [pltpu.SemaphoreType.DMA] * (2 * BUFN)
        ),
        compiler_params=pltpu.CompilerParams(use_tc_tiling_on_sc=False),
    )


def _ln(x, g, b):
    m = jnp.mean(x, axis=-1, keepdims=True)
    v = jnp.mean((x - m) ** 2, axis=-1, keepdims=True)
    return (x - m) * lax.rsqrt(v + 1e-5) * g + b


def _dis(degp):
    deg = degp[0][:, None] + degp[1][:, None] + 1.0
    return lax.rsqrt(jnp.maximum(deg, 1.0))


def _cnt(cntp):
    return jnp.maximum(cntp[0][:, None] + cntp[1][:, None], 1.0)


def _pvec():
    return pl.BlockSpec((NC, RB), lambda i: (0, i))


def _rows(f):
    return pl.BlockSpec((RB, f), lambda i: (i, 0))


def _full(a, b):
    return pl.BlockSpec((a, b), lambda i: (0, 0))


def _parts(f):
    return pl.BlockSpec((NC, RB, f), lambda i: (0, i, 0))


def _out2(f):
    return jax.ShapeDtypeStruct((NP, f), jnp.float32)


def _in0_body(x_ref, w0_ref, rw_ref, rg_ref, rb_ref, degp_ref,
              hw_ref, h2_ref, x0_ref):
    xb = x_ref[...]
    hw = jnp.dot(xb, w0_ref[...], preferred_element_type=jnp.float32)
    dis = _dis(degp_ref)
    hw_ref[...] = hw
    h2_ref[...] = hw * dis
    r = jnp.maximum(jnp.dot(xb, rw_ref[...], preferred_element_type=jnp.float32), 0.0)
    x0_ref[...] = _ln(r, rg_ref[...], rb_ref[...])


@functools.lru_cache(maxsize=None)
def _make_in0(d):
    return pl.pallas_call(
        _in0_body,
        grid=(GRID,),
        in_specs=[_rows(d), _full(d, H), _full(d, H), _full(1, H), _full(1, H),
                  _pvec()],
        out_specs=[_rows(H), _rows(H), _rows(H)],
        out_shape=[_out2(H), _out2(H), _out2(H)],
    )


def _conv_out_body(aggp_ref, hw_ref, degp_ref, b_ref, g_ref, lb_ref, out_ref):
    agg = aggp_ref[0] + aggp_ref[1]
    dis = _dis(degp_ref)
    o = dis * (agg + dis * hw_ref[...]) + b_ref[...]
    out_ref[...] = _ln(jnp.maximum(o, 0.0), g_ref[...], lb_ref[...])


@functools.lru_cache(maxsize=None)
def _make_conv_out():
    return pl.pallas_call(
        _conv_out_body,
        grid=(GRID,),
        in_specs=[_parts(H), _rows(H), _pvec(), _full(1, H), _full(1, H),
                  _full(1, H)],
        out_specs=_rows(H),
        out_shape=_out2(H),
    )


def _conv_final_body(aggp_ref, hw_ref, degp_ref, b_ref, out_ref):
    agg = aggp_ref[0] + aggp_ref[1]
    dis = _dis(degp_ref)
    out_ref[...] = dis * (agg + dis * hw_ref[...]) + b_ref[...]


@functools.lru_cache(maxsize=None)
def _make_conv_final(w):
    return pl.pallas_call(
        _conv_final_body,
        grid=(GRID,),
        in_specs=[_parts(w), _rows(w), _pvec(), _full(1, w)],
        out_specs=_rows(w),
        out_shape=_out2(w),
    )


def _mean_body(s1p_ref, cntp_ref, h_ref, xm_ref, t_ref):
    cnt = _cnt(cntp_ref)
    xm = (s1p_ref[0] + s1p_ref[1]) / cnt
    xm_ref[...] = xm
    t_ref[...] = jnp.abs(h_ref[...] - xm)


@functools.lru_cache(maxsize=None)
def _make_mean():
    return pl.pallas_call(
        _mean_body,
        grid=(GRID,),
        in_specs=[_parts(H), _pvec(), _rows(H)],
        out_specs=[_rows(H), _rows(H)],
        out_shape=[_out2(H), _out2(H)],
    )


def _gate_body(s2p_ref, cntp_ref, h_ref, xm_ref, x0_ref, wsa_ref, wsb_ref,
               wsc_ref, wn_ref, degp_ref, hw_ref, h2_ref):
    cnt = _cnt(cntp_ref)
    xs = (s2p_ref[0] + s2p_ref[1]) / cnt
    h = h_ref[...]
    xm = xm_ref[...]
    logit = (jnp.sum(xm * h * wsa_ref[...], axis=-1, keepdims=True)
             + jnp.sum(xs * wsb_ref[...], axis=-1, keepdims=True)
             + jnp.sum(h * wsc_ref[...], axis=-1, keepdims=True))
    score = jax.nn.sigmoid(logit)
    hn = (1.0 - score) * h + score * x0_ref[...]
    hw = jnp.dot(hn, wn_ref[...], preferred_element_type=jnp.float32)
    dis = _dis(degp_ref)
    hw_ref[...] = hw
    h2_ref[...] = hw * dis


@functools.lru_cache(maxsize=None)
def _make_gate(wo):
    return pl.pallas_call(
        _gate_body,
        grid=(GRID,),
        in_specs=[_parts(H), _pvec(), _rows(H), _rows(H), _rows(H),
                  _full(1, H), _full(1, H), _full(1, H), _full(H, wo),
                  _pvec()],
        out_specs=[_rows(wo), _rows(wo)],
        out_shape=[_out2(wo), _out2(wo)],
    )


def kernel(x, edge_index, W0, b0, W1, b1, W2, b2, W3, b3, ln_g0, ln_b0,
           ln_g1, ln_b1, ln_g2, ln_b2, ws_W, res_W, res_ln_g, res_ln_b):
    n, d = x.shape
    e = edge_index.shape[1]
    ec = -(-e // (NW * 8 * CHUNK)) * NW * 8
    epad = ec * CHUNK - e
    padrow = NP - 8

    src = edge_index[0].astype(jnp.int32)
    dst = edge_index[1].astype(jnp.int32)
    src_g = jnp.pad(src, (0, epad)).reshape(ec, CHUNK)
    dst_g = jnp.pad(dst, (0, epad)).reshape(ec, CHUNK)
    src_s = jnp.pad(src, (0, epad), constant_values=padrow).reshape(ec, CHUNK)
    dst_s = jnp.pad(dst, (0, epad), constant_values=padrow).reshape(ec, CHUNK)

    xp = jnp.pad(x, ((0, NP - n), (0, 0)))
    ident = jnp.arange(HR, dtype=jnp.int32).reshape(HR // CHUNK, CHUNK)
    zerosH = jnp.zeros((NP, H), jnp.float32)

    c = W3.shape[1]
    w3 = 48
    r2 = lambda a: a.reshape(1, H)
    wsa = ws_W[0:H, 0].reshape(1, H)
    wsb = ws_W[H:2 * H, 0].reshape(1, H)
    wsc = ws_W[2 * H:3 * H, 0].reshape(1, H)
    w3p = jnp.pad(W3, ((0, 0), (0, w3 - c)))
    b3p = jnp.pad(b3, (0, w3 - c)).reshape(1, w3)
    zeros3 = jnp.zeros((NP, w3), jnp.float32)

    counts = _make_counts(ec)
    agg = _make_agg(ec)
    agg3 = _make_agg(ec, w3)
    conv_out = _make_conv_out()
    mean = _make_mean()

    degp, cntp = counts(dst_s, src_s, ident)
    degp = degp.reshape(NC, NP)
    cntp = cntp.reshape(NC, NP)

    hw, h2, x0 = _make_in0(d)(xp, W0, res_W, r2(res_ln_g), r2(res_ln_b), degp)
    aggp = agg(h2, src_g, dst_s, zerosH)
    h = conv_out(aggp, hw, degp, r2(b0), r2(ln_g0), r2(ln_b0))

    hs = [h]
    gs = [ln_g1, ln_g2]
    lbs = [ln_b1, ln_b2]
    bs = [b1, b2]
    wn = [W1, W2, w3p]
    for i in range(3):
        s1p = agg(h, dst_g, src_s, zerosH)
        xm, t = mean(s1p, cntp, h)
        s2p = agg(t, dst_g, src_s, zerosH)
        gate = _make_gate(wn[i].shape[1])
        hw, h2 = gate(s2p, cntp, h, xm, x0, wsa, wsb, wsc, wn[i], degp)
        if i < 2:
            aggp = agg(h2, src_g, dst_s, zerosH)
            h = conv_out(aggp, hw, degp, r2(bs[i]), r2(gs[i]), r2(lbs[i]))
            hs.append(h)
        else:
            aggp = agg3(h2, src_g, dst_s, zeros3)
            out = _make_conv_final(w3)(aggp, hw, degp, b3p)

    return (out[:n, :c], hs[0][:n], hs[1][:n], hs[2][:n])


# speedup vs baseline: 1.0099x; 1.0099x over previous
import functools

import jax
import jax.numpy as jnp
from jax import lax
from jax.experimental import pallas as pl
from jax.experimental.pallas import tpu as pltpu
from jax.experimental.pallas import tpu_sc as plsc

N = 10000
NP = 10240
H = 64
RB = 512
GRID = NP // RB
CHUNK = 128
NC = 2
NS = 16
NW = NC * NS
RPS = NP // NS


HR = NP // 16


def _counts_body(dst_hbm, src_hbm, ident_hbm, degp, cntp,
                 dsti_v, srci_v, ident_v, dh, ch, dacc, cacc):
    cid = lax.axis_index("c")
    sid = lax.axis_index("s")
    w = cid * NS + sid
    cpw = dst_hbm.shape[0] // NW
    zv = jnp.zeros((16,), jnp.float32)

    def zero(r, c):
        dh[r] = zv
        ch[r] = zv
        return c

    lax.fori_loop(0, HR, zero, 0)

    @pl.when(sid == 0)
    def _():
        pltpu.sync_copy(dh, dacc)
        pltpu.sync_copy(ch, cacc)

    pltpu.sync_copy(ident_hbm, ident_v)
    pltpu.sync_copy(dst_hbm.at[pl.ds(w * cpw, cpw)], dsti_v)
    pltpu.sync_copy(src_hbm.at[pl.ds(w * cpw, cpw)], srci_v)
    plsc.subcore_barrier()

    ones = jnp.ones((16,), jnp.float32)

    def hist(k, c):
        for u in range(CHUNK // 16):
            di = dsti_v[k, pl.ds(u * 16, 16)]
            plsc.addupdate_scatter(dh, [di >> 4, di & 15], ones)
            si = srci_v[k, pl.ds(u * 16, 16)]
            plsc.addupdate_scatter(ch, [si >> 4, si & 15], ones)
        return c

    lax.fori_loop(0, cpw, hist, 0)
    for k in range(HR // CHUNK):
        s = pl.ds(k * CHUNK, CHUNK)
        pltpu.sync_copy(dh.at[s], dacc.at[ident_v.at[k]], add=True)
        pltpu.sync_copy(ch.at[s], cacc.at[ident_v.at[k]], add=True)
    plsc.subcore_barrier()
    rps = HR // NS
    s = pl.ds(sid * rps, rps)
    pltpu.sync_copy(dacc.at[s], degp.at[cid, s])
    pltpu.sync_copy(cacc.at[s], cntp.at[cid, s])


@functools.lru_cache(maxsize=None)
def _make_counts(ec_rows):
    mesh = plsc.VectorSubcoreMesh(core_axis_name="c", subcore_axis_name="s",
                                  num_cores=NC, num_subcores=NS)
    cpw = ec_rows // NW
    return pl.kernel(
        _counts_body,
        out_type=(jax.ShapeDtypeStruct((NC, HR, 16), jnp.float32),
                  jax.ShapeDtypeStruct((NC, HR, 16), jnp.float32)),
        mesh=mesh,
        scratch_types=[
            pltpu.VMEM((cpw, CHUNK), jnp.int32),
            pltpu.VMEM((cpw, CHUNK), jnp.int32),
            pltpu.VMEM((HR // CHUNK, CHUNK), jnp.int32),
            pltpu.VMEM((HR, 16), jnp.float32),
            pltpu.VMEM((HR, 16), jnp.float32),
            pltpu.VMEM_SHARED((HR, 16), jnp.float32),
            pltpu.VMEM_SHARED((HR, 16), jnp.float32),
        ],
        compiler_params=pltpu.CompilerParams(use_tc_tiling_on_sc=False,
                                             needs_layout_passes=False),
    )


BUFN = 4
LEAD = BUFN // 2


def _agg_body(table_hbm, gi_hbm, si_hbm, zeros_hbm, aggp, gi_v, si_v, *sc):
    bufs = sc[:BUFN]
    acc = sc[BUFN]
    gsem = sc[BUFN + 1:2 * BUFN + 1]
    ssem = sc[2 * BUFN + 1:]
    cid = lax.axis_index("c")
    sid = lax.axis_index("s")
    w = cid * NS + sid
    cpw = gi_hbm.shape[0] // NW
    r0 = sid * RPS
    pltpu.sync_copy(zeros_hbm.at[pl.ds(r0, RPS)], acc.at[pl.ds(r0, RPS)])
    pltpu.sync_copy(gi_hbm.at[pl.ds(w * cpw, cpw)], gi_v)
    pltpu.sync_copy(si_hbm.at[pl.ds(w * cpw, cpw)], si_v)
    plsc.subcore_barrier()

    def gather(j, b):
        pltpu.async_copy(table_hbm.at[gi_v.at[j]], bufs[b], gsem[b])

    def gather_wait(j, b):
        pltpu.make_async_copy(table_hbm.at[gi_v.at[j]], bufs[b], gsem[b]).wait()

    def scat(j, b):
        pltpu.async_copy(bufs[b], acc.at[si_v.at[j]], ssem[b], add=True)

    def scat_wait(j, b):
        pltpu.make_async_copy(bufs[b], acc.at[si_v.at[j]], ssem[b]).wait()

    for b in range(LEAD):
        gather(b, b)
    for b in range(BUFN):
        gather_wait(b, b)
        scat(b, b)
        if b >= LEAD:
            scat_wait(b - LEAD, (b + LEAD) % BUFN)
        gather(b + LEAD, (b + LEAD) % BUFN)

    def body(jr, c):
        j0 = jr * BUFN
        for b in range(BUFN):
            jb = j0 + b
            gather_wait(jb, b)
            scat(jb, b)
            scat_wait(jb - LEAD, (b + LEAD) % BUFN)
            gather(lax.rem(jb + LEAD, cpw), (b + LEAD) % BUFN)
        return c

    lax.fori_loop(1, cpw // BUFN, body, 0)
    for t in range(LEAD):
        scat_wait(cpw - LEAD + t, (cpw - LEAD + t) % BUFN)
        gather_wait(t, t)
    plsc.subcore_barrier()
    pltpu.sync_copy(acc.at[pl.ds(r0, RPS)], aggp.at[cid, pl.ds(r0, RPS)])


@functools.lru_cache(maxsize=None)
def _make_agg(ec_rows, w=H):
    mesh = plsc.VectorSubcoreMesh(core_axis_name="c", subcore_axis_name="s",
                                  num_cores=NC, num_subcores=NS)
    cpw = ec_rows // NW
    return pl.kernel(
        _agg_body,
        out_type=jax.ShapeDtypeStruct((NC, NP, w), jnp.float32),
        mesh=mesh,
        scratch_types=(
            [pltpu.VMEM((cpw, CHUNK), jnp.int32)] * 2
            + [pltpu.VMEM((CHUNK, w), jnp.float32)] * BUFN
            + [pltpu.VMEM_SHARED((NP, w), jnp.float32)]
            + [pltpu.SemaphoreType.DMA] * (2 * BUFN)
        ),
        compiler_params=pltpu.CompilerParams(use_tc_tiling_on_sc=False),
    )


def _ln(x, g, b):
    m = jnp.mean(x, axis=-1, keepdims=True)
    v = jnp.mean((x - m) ** 2, axis=-1, keepdims=True)
    return (x - m) * lax.rsqrt(v + 1e-5) * g + b


def _dis(degp):
    deg = degp[0][:, None] + degp[1][:, None] + 1.0
    return lax.rsqrt(jnp.maximum(deg, 1.0))


def _cnt(cntp):
    return jnp.maximum(cntp[0][:, None] + cntp[1][:, None], 1.0)


def _pvec():
    return pl.BlockSpec((NC, RB), lambda i: (0, i))


def _rows(f):
    return pl.BlockSpec((RB, f), lambda i: (i, 0))


def _full(a, b):
    return pl.BlockSpec((a, b), lambda i: (0, 0))


def _parts(f):
    return pl.BlockSpec((NC, RB, f), lambda i: (0, i, 0))


def _out2(f):
    return jax.ShapeDtypeStruct((NP, f), jnp.float32)


def _in0_body(x_ref, w0_ref, rw_ref, rg_ref, rb_ref, degp_ref,
              hw_ref, h2_ref, x0_ref):
    xb = x_ref[...]
    hw = jnp.dot(xb, w0_ref[...], preferred_element_type=jnp.float32)
    dis = _dis(degp_ref)
    hw_ref[...] = hw
    h2_ref[...] = hw * dis
    r = jnp.maximum(jnp.dot(xb, rw_ref[...], preferred_element_type=jnp.float32), 0.0)
    x0_ref[...] = _ln(r, rg_ref[...], rb_ref[...])


@functools.lru_cache(maxsize=None)
def _make_in0(d):
    return pl.pallas_call(
        _in0_body,
        grid=(GRID,),
        in_specs=[_rows(d), _full(d, H), _full(d, H), _full(1, H), _full(1, H),
                  _pvec()],
        out_specs=[_rows(H), _rows(H), _rows(H)],
        out_shape=[_out2(H), _out2(H), _out2(H)],
    )


def _conv_out_body(aggp_ref, hw_ref, degp_ref, b_ref, g_ref, lb_ref, out_ref):
    agg = aggp_ref[0] + aggp_ref[1]
    dis = _dis(degp_ref)
    o = dis * (agg + dis * hw_ref[...]) + b_ref[...]
    out_ref[...] = _ln(jnp.maximum(o, 0.0), g_ref[...], lb_ref[...])


@functools.lru_cache(maxsize=None)
def _make_conv_out():
    return pl.pallas_call(
        _conv_out_body,
        grid=(GRID,),
        in_specs=[_parts(H), _rows(H), _pvec(), _full(1, H), _full(1, H),
                  _full(1, H)],
        out_specs=_rows(H),
        out_shape=_out2(H),
    )


def _conv_final_body(aggp_ref, hw_ref, degp_ref, b_ref, out_ref):
    agg = aggp_ref[0] + aggp_ref[1]
    dis = _dis(degp_ref)
    out_ref[...] = dis * (agg + dis * hw_ref[...]) + b_ref[...]


@functools.lru_cache(maxsize=None)
def _make_conv_final(w):
    return pl.pallas_call(
        _conv_final_body,
        grid=(GRID,),
        in_specs=[_parts(w), _rows(w), _pvec(), _full(1, w)],
        out_specs=_rows(w),
        out_shape=_out2(w),
    )


def _mean_body(s1p_ref, cntp_ref, h_ref, xm_ref, t_ref):
    cnt = _cnt(cntp_ref)
    xm = (s1p_ref[0] + s1p_ref[1]) / cnt
    xm_ref[...] = xm
    t_ref[...] = jnp.abs(h_ref[...] - xm)


@functools.lru_cache(maxsize=None)
def _make_mean():
    return pl.pallas_call(
        _mean_body,
        grid=(GRID,),
        in_specs=[_parts(H), _pvec(), _rows(H)],
        out_specs=[_rows(H), _rows(H)],
        out_shape=[_out2(H), _out2(H)],
    )


def _gate_body(s2p_ref, cntp_ref, h_ref, xm_ref, x0_ref, wsa_ref, wsb_ref,
               wsc_ref, wn_ref, degp_ref, hw_ref, h2_ref):
    cnt = _cnt(cntp_ref)
    xs = (s2p_ref[0] + s2p_ref[1]) / cnt
    h = h_ref[...]
    xm = xm_ref[...]
    logit = (jnp.sum(xm * h * wsa_ref[...], axis=-1, keepdims=True)
             + jnp.sum(xs * wsb_ref[...], axis=-1, keepdims=True)
             + jnp.sum(h * wsc_ref[...], axis=-1, keepdims=True))
    score = jax.nn.sigmoid(logit)
    hn = (1.0 - score) * h + score * x0_ref[...]
    hw = jnp.dot(hn, wn_ref[...], preferred_element_type=jnp.float32)
    dis = _dis(degp_ref)
    hw_ref[...] = hw
    h2_ref[...] = hw * dis


@functools.lru_cache(maxsize=None)
def _make_gate(wo):
    return pl.pallas_call(
        _gate_body,
        grid=(GRID,),
        in_specs=[_parts(H), _pvec(), _rows(H), _rows(H), _rows(H),
                  _full(1, H), _full(1, H), _full(1, H), _full(H, wo),
                  _pvec()],
        out_specs=[_rows(wo), _rows(wo)],
        out_shape=[_out2(wo), _out2(wo)],
    )


def kernel(x, edge_index, W0, b0, W1, b1, W2, b2, W3, b3, ln_g0, ln_b0,
           ln_g1, ln_b1, ln_g2, ln_b2, ws_W, res_W, res_ln_g, res_ln_b):
    n, d = x.shape
    e = edge_index.shape[1]
    ec = -(-e // (NW * 8 * CHUNK)) * NW * 8
    epad = ec * CHUNK - e
    padrow = NP - 8

    src = edge_index[0].astype(jnp.int32)
    dst = edge_index[1].astype(jnp.int32)
    src_g = jnp.pad(src, (0, epad)).reshape(ec, CHUNK)
    dst_g = jnp.pad(dst, (0, epad)).reshape(ec, CHUNK)
    src_s = jnp.pad(src, (0, epad), constant_values=padrow).reshape(ec, CHUNK)
    dst_s = jnp.pad(dst, (0, epad), constant_values=padrow).reshape(ec, CHUNK)

    xp = jnp.pad(x, ((0, NP - n), (0, 0)))
    ident = jnp.arange(HR, dtype=jnp.int32).reshape(HR // CHUNK, CHUNK)
    zerosH = jnp.zeros((NP, H), jnp.float32)

    c = W3.shape[1]
    w3 = 48
    r2 = lambda a: a.reshape(1, H)
    wsa = ws_W[0:H, 0].reshape(1, H)
    wsb = ws_W[H:2 * H, 0].reshape(1, H)
    wsc = ws_W[2 * H:3 * H, 0].reshape(1, H)
    w3p = jnp.pad(W3, ((0, 0), (0, w3 - c)))
    b3p = jnp.pad(b3, (0, w3 - c)).reshape(1, w3)
    zeros3 = jnp.zeros((NP, w3), jnp.float32)

    counts = _make_counts(ec)
    agg = _make_agg(ec)
    agg3 = _make_agg(ec, w3)
    conv_out = _make_conv_out()
    mean = _make_mean()

    degp, cntp = counts(dst_s, src_s, ident)
    degp = degp.reshape(NC, NP)
    cntp = cntp.reshape(NC, NP)

    hw, h2, x0 = _make_in0(d)(xp, W0, res_W, r2(res_ln_g), r2(res_ln_b), degp)
    aggp = agg(h2, src_g, dst_s, zerosH)
    h = conv_out(aggp, hw, degp, r2(b0), r2(ln_g0), r2(ln_b0))

    hs = [h]
    gs = [ln_g1, ln_g2]
    lbs = [ln_b1, ln_b2]
    bs = [b1, b2]
    wn = [W1, W2, w3p]
    for i in range(3):
        s1p = agg(h, dst_g, src_s, zerosH)
        xm, t = mean(s1p, cntp, h)
        s2p = agg(t, dst_g, src_s, zerosH)
        gate = _make_gate(wn[i].shape[1])
        hw, h2 = gate(s2p, cntp, h, xm, x0, wsa, wsb, wsc, wn[i], degp)
        if i < 2:
            aggp = agg(h2, src_g, dst_s, zerosH)
            h = conv_out(aggp, hw, degp, r2(bs[i]), r2(gs[i]), r2(lbs[i]))
            hs.append(h)
        else:
            aggp = agg3(h2, src_g, dst_s, zeros3)
            out = _make_conv_final(w3)(aggp, hw, degp, b3p)

    return (out[:n, :c], hs[0][:n], hs[1][:n], hs[2][:n])
